# Initial kernel scaffold; baseline (speedup 1.0000x reference)
import jax
import jax.numpy as jnp
from jax.experimental import pallas as pl


def kernel(xyz, bid):
    raise NotImplementedError("write your pallas kernel here")


# speedup vs baseline: 1.9749x; 1.9749x over previous
import jax
import jax.numpy as jnp
from jax.experimental import pallas as pl
from jax.experimental.pallas import tpu as pltpu

_GRID = 0.05
_ROWS = 8192
_LANES = 128
_BLK = 1024


def _spread10(v):
    v = v & 0x3FF
    v = (v | (v << 16)) & 0x30000FF
    v = (v | (v << 8)) & 0x300F00F
    v = (v | (v << 4)) & 0x30C30C3
    v = (v | (v << 2)) & 0x9249249
    return v


def _morton_body(gx_ref, gy_ref, gz_ref, out_ref):
    x = gx_ref[...]
    y = gy_ref[...]
    z = gz_ref[...]
    out_ref[...] = _spread10(x) | (_spread10(y) << 1) | (_spread10(z) << 2)


def _morton_codes(gx, gy, gz):
    return pl.pallas_call(
        _morton_body,
        out_shape=jax.ShapeDtypeStruct((_ROWS, _LANES), jnp.int32),
    )(gx, gy, gz)


def kernel(xyz, bid):
    b, n, _ = xyz.shape
    flat = xyz.reshape(b * n, 3)
    mn = jnp.min(flat, axis=0, keepdims=True)
    g = jnp.floor((flat - mn) / _GRID).astype(jnp.int32)
    gx = g[:, 0].reshape(_ROWS, _LANES)
    gy = g[:, 1].reshape(_ROWS, _LANES)
    gz = g[:, 2].reshape(_ROWS, _LANES)
    codes = _morton_codes(gx, gy, gz).reshape(b, n)
    idx = jnp.argsort(codes, axis=1, stable=True)
    return idx.astype(jnp.int64)


# speedup vs baseline: 4.7862x; 2.4235x over previous
import functools

import jax
import jax.numpy as jnp
from jax import lax
from jax.experimental import pallas as pl
from jax.experimental.pallas import tpu as pltpu
from jax.experimental.pallas import tpu_sc as plsc

_GRID = 0.05
_B = 8
_N = 131072
_ROWS = 8192
_LANES = 128

_NC = 2
_NT = 16
_CHUNK = _N // _NT
_BPC = _B // _NC
_R = 1024
_VPC = _CHUNK // 16

I32 = jnp.int32


def _spread10(v):
    v = v & 0x3FF
    v = (v | (v << 16)) & 0x30000FF
    v = (v | (v << 8)) & 0x300F00F
    v = (v | (v << 4)) & 0x30C30C3
    v = (v | (v << 2)) & 0x9249249
    return v


def _morton_body(gx_ref, gy_ref, gz_ref, out_ref):
    x = gx_ref[...]
    y = gy_ref[...]
    z = gz_ref[...]
    out_ref[...] = _spread10(x) | (_spread10(y) << 1) | (_spread10(z) << 2)


def _morton_codes(gx, gy, gz):
    return pl.pallas_call(
        _morton_body,
        out_shape=jax.ShapeDtypeStruct((_ROWS, _LANES), jnp.int32),
    )(gx, gy, gz)


def _sort_body(codes_hbm, perm_hbm, kv, vv, pv, hist, offs, gbuf,
               ska, sva, skb, svb, sgrid, sem, sem2):
    c = lax.axis_index("c")
    s = lax.axis_index("s")
    iota = lax.iota(I32, 16)
    zeros16 = jnp.zeros((16,), I32)
    cbase = s * _CHUNK

    def histogram(shift):
        def zero_body(j, _):
            hist[pl.ds(j * 16, 16)] = zeros16
            return 0
        lax.fori_loop(I32(0), I32(_R // 16), zero_body, 0)

        def hist_body(j, _):
            k = kv[pl.ds(j * 16, 16)]
            d = (k >> shift) & (_R - 1)
            cnt, last = plsc.scan_count(d)
            old = plsc.load_gather(hist, [d])
            plsc.store_scatter(hist, [d], old + cnt, mask=last)
            return 0
        lax.fori_loop(I32(0), I32(_VPC), hist_body, 0)

    def offsets():
        pltpu.sync_copy(sgrid, gbuf)

        def grp_body(j, carry):
            tot = zeros16
            bef = zeros16
            for t in range(_NT):
                row = gbuf[t, pl.ds(j * 16, 16)]
                bef = jnp.where(jnp.full((16,), t, I32) < s, bef + row, bef)
                tot = tot + row
            ex = plsc.cumsum(tot) - tot
            offs[pl.ds(j * 16, 16)] = carry + ex + bef
            return carry + jnp.sum(tot, dtype=I32)
        lax.fori_loop(I32(0), I32(_R // 16), grp_body, jnp.zeros((), I32))

    def rank_permute(shift, first):
        def rp_body(j, _):
            k = kv[pl.ds(j * 16, 16)]
            d = (k >> shift) & (_R - 1)
            cnt, last = plsc.scan_count(d)
            old = plsc.load_gather(offs, [d])
            pv[pl.ds(j * 16, 16)] = old + cnt - 1
            plsc.store_scatter(offs, [d], old + cnt, mask=last)
            if first:
                vv[pl.ds(j * 16, 16)] = cbase + j * 16 + iota
            return 0
        lax.fori_loop(I32(0), I32(_VPC), rp_body, 0)

    for bi in range(_BPC):
        b = c * _BPC + bi
        hbase = b * _N + cbase

        pltpu.sync_copy(codes_hbm.at[pl.ds(hbase, _CHUNK)], kv)
        histogram(0)
        pltpu.sync_copy(hist, sgrid.at[s])
        plsc.subcore_barrier()
        offsets()
        rank_permute(0, first=True)
        cp1 = pltpu.async_copy(kv, ska.at[pv], sem)
        cp2 = pltpu.async_copy(vv, sva.at[pv], sem2)
        cp1.wait()
        cp2.wait()
        plsc.subcore_barrier()

        pltpu.sync_copy(ska.at[pl.ds(cbase, _CHUNK)], kv)
        histogram(10)
        pltpu.sync_copy(hist, sgrid.at[s])
        plsc.subcore_barrier()
        offsets()
        pltpu.sync_copy(sva.at[pl.ds(cbase, _CHUNK)], vv)
        rank_permute(10, first=False)
        cp1 = pltpu.async_copy(kv, skb.at[pv], sem)
        cp2 = pltpu.async_copy(vv, svb.at[pv], sem2)
        cp1.wait()
        cp2.wait()
        plsc.subcore_barrier()

        pltpu.sync_copy(skb.at[pl.ds(cbase, _CHUNK)], kv)
        histogram(20)
        pltpu.sync_copy(hist, sgrid.at[s])
        plsc.subcore_barrier()
        offsets()
        pltpu.sync_copy(svb.at[pl.ds(cbase, _CHUNK)], vv)
        rank_permute(20, first=False)
        pltpu.async_copy(vv, sva.at[pv], sem).wait()
        plsc.subcore_barrier()

        pltpu.sync_copy(sva.at[pl.ds(cbase, _CHUNK)], vv)
        pltpu.sync_copy(vv, perm_hbm.at[pl.ds(hbase, _CHUNK)])


def _sc_sort(codes_flat):
    mesh = plsc.VectorSubcoreMesh(
        core_axis_name="c", subcore_axis_name="s", num_cores=_NC
    )
    return pl.kernel(
        _sort_body,
        out_type=jax.ShapeDtypeStruct((_B * _N,), I32),
        mesh=mesh,
        compiler_params=pltpu.CompilerParams(needs_layout_passes=False),
        scratch_types=[
            pltpu.VMEM((_CHUNK,), I32),
            pltpu.VMEM((_CHUNK,), I32),
            pltpu.VMEM((_CHUNK,), I32),
            pltpu.VMEM((_R,), I32),
            pltpu.VMEM((_R,), I32),
            pltpu.VMEM((_NT, _R), I32),
            pltpu.VMEM_SHARED((_N,), I32),
            pltpu.VMEM_SHARED((_N,), I32),
            pltpu.VMEM_SHARED((_N,), I32),
            pltpu.VMEM_SHARED((_N,), I32),
            pltpu.VMEM_SHARED((_NT, _R), I32),
            pltpu.SemaphoreType.DMA,
            pltpu.SemaphoreType.DMA,
        ],
    )(codes_flat)


def kernel(xyz, bid):
    b, n, _ = xyz.shape
    flat = xyz.reshape(b * n, 3)
    mn = jnp.min(flat, axis=0, keepdims=True)
    g = jnp.floor((flat - mn) / _GRID).astype(jnp.int32)
    gx = g[:, 0].reshape(_ROWS, _LANES)
    gy = g[:, 1].reshape(_ROWS, _LANES)
    gz = g[:, 2].reshape(_ROWS, _LANES)
    codes = _morton_codes(gx, gy, gz).reshape(b * n)
    perm = _sc_sort(codes)
    return perm.reshape(b, n).astype(jnp.int64)
